# baseline (speedup 1.0000x reference)
import functools

import jax
import jax.numpy as jnp
from jax import lax
from jax.experimental import pallas as pl
from jax.experimental.pallas import tpu as pltpu, tpu_sc as plsc

N = 1024
D = 768
N_SC = 128
N_TC = N - N_SC
BLK = 128

_info = plsc.get_sparse_core_info()
_NC = _info.num_cores
_NS = _info.num_subcores
_NW = _NC * _NS
_RPW = N_SC // _NW


@functools.partial(
    pl.kernel,
    mesh=plsc.VectorSubcoreMesh(core_axis_name="c", subcore_axis_name="s"),
    out_type=jax.ShapeDtypeStruct((N_SC, D), jnp.float32),
    scratch_types=[pltpu.VMEM((_RPW, D), jnp.float32)],
)
def _sc_gather_tail(pe_hbm, out_hbm, buf):
    wid = lax.axis_index("s") * _NC + lax.axis_index("c")
    base = wid * _RPW
    pltpu.sync_copy(pe_hbm.at[pl.ds(N_TC + base, _RPW)], buf)
    pltpu.sync_copy(buf, out_hbm.at[pl.ds(base, _RPW)])


def _copy_body(pe_ref, out_ref):
    out_ref[...] = pe_ref[...]


def _tc_copy_head(pe):
    return pl.pallas_call(
        _copy_body,
        grid=(N_TC // BLK,),
        in_specs=[pl.BlockSpec((BLK, D), lambda i: (i, 0))],
        out_specs=pl.BlockSpec((BLK, D), lambda i: (i, 0)),
        out_shape=jax.ShapeDtypeStruct((N, D), jnp.float32),
    )(pe)


def kernel(h, w, pe):
    tail = _sc_gather_tail(pe)
    full = _tc_copy_head(pe)
    out = lax.dynamic_update_slice(full, tail, (N_TC, 0))
    return out[None]


# speedup vs baseline: 1.0809x; 1.0809x over previous
import functools

import jax
import jax.numpy as jnp
from jax import lax
from jax.experimental import pallas as pl
from jax.experimental.pallas import tpu as pltpu, tpu_sc as plsc

N = 1024
D = 768

_info = plsc.get_sparse_core_info()
_NC = _info.num_cores
_NS = _info.num_subcores
_NW = _NC * _NS
_RPW = N // _NW
_HALF = _RPW // 2


@functools.partial(
    pl.kernel,
    mesh=plsc.VectorSubcoreMesh(core_axis_name="c", subcore_axis_name="s"),
    out_type=jax.ShapeDtypeStruct((N, D), jnp.float32),
    scratch_types=[
        pltpu.VMEM((_HALF, D), jnp.float32),
        pltpu.VMEM((_HALF, D), jnp.float32),
        pltpu.SemaphoreType.DMA,
        pltpu.SemaphoreType.DMA,
        pltpu.SemaphoreType.DMA,
        pltpu.SemaphoreType.DMA,
    ],
)
def _pe_copy(pe_hbm, out_hbm, buf0, buf1, r0, r1, w0, w1):
    wid = lax.axis_index("s") * _NC + lax.axis_index("c")
    base = wid * _RPW
    rd0 = pltpu.async_copy(pe_hbm.at[pl.ds(base, _HALF)], buf0, r0)
    rd1 = pltpu.async_copy(pe_hbm.at[pl.ds(base + _HALF, _HALF)], buf1, r1)
    rd0.wait()
    wr0 = pltpu.async_copy(buf0, out_hbm.at[pl.ds(base, _HALF)], w0)
    rd1.wait()
    wr1 = pltpu.async_copy(buf1, out_hbm.at[pl.ds(base + _HALF, _HALF)], w1)
    wr0.wait()
    wr1.wait()


def kernel(h, w, pe):
    return _pe_copy(pe)[None]


# speedup vs baseline: 1.0817x; 1.0008x over previous
import functools

import jax
import jax.numpy as jnp
from jax import lax
from jax.experimental import pallas as pl
from jax.experimental.pallas import tpu as pltpu, tpu_sc as plsc

N = 1024
D = 768

_NC = 2
_NS = 16
_NW = _NC * _NS
_RPW = N // _NW
_HALF = _RPW // 2


@functools.partial(
    pl.kernel,
    mesh=plsc.VectorSubcoreMesh(
        core_axis_name="c", subcore_axis_name="s",
        num_cores=_NC, num_subcores=_NS),
    out_type=jax.ShapeDtypeStruct((N, D), jnp.float32),
    scratch_types=[
        pltpu.VMEM((_HALF, D), jnp.float32),
        pltpu.VMEM((_HALF, D), jnp.float32),
        pltpu.SemaphoreType.DMA,
        pltpu.SemaphoreType.DMA,
        pltpu.SemaphoreType.DMA,
        pltpu.SemaphoreType.DMA,
    ],
)
def _pe_copy(pe_hbm, out_hbm, buf0, buf1, r0, r1, w0, w1):
    wid = lax.axis_index("s") * _NC + lax.axis_index("c")
    base = wid * _RPW
    rd0 = pltpu.async_copy(pe_hbm.at[pl.ds(base, _HALF)], buf0, r0)
    rd1 = pltpu.async_copy(pe_hbm.at[pl.ds(base + _HALF, _HALF)], buf1, r1)
    rd0.wait()
    wr0 = pltpu.async_copy(buf0, out_hbm.at[pl.ds(base, _HALF)], w0)
    rd1.wait()
    wr1 = pltpu.async_copy(buf1, out_hbm.at[pl.ds(base + _HALF, _HALF)], w1)
    wr0.wait()
    wr1.wait()


def kernel(h, w, pe):
    return _pe_copy(pe)[None]
